# Initial kernel scaffold; baseline (speedup 1.0000x reference)
import functools

import jax
import jax.numpy as jnp
from jax import lax
from jax.experimental import pallas as pl
from jax.experimental.pallas import tpu as pltpu
from jax.experimental.pallas import tpu_sc as plsc

N = 10000
E = 320000
D = 128
NPAD = 10016
NW = 32
EPT = E // NW
CH = 128
NCHUNK = 80
EPTP = NCHUNK * CH
RPT = NPAD // 16
R = 1000
F32 = jnp.float32

_mesh = plsc.VectorSubcoreMesh(core_axis_name="c", subcore_axis_name="s")


@functools.partial(
    pl.kernel,
    out_type=jax.ShapeDtypeStruct((2, NPAD, 16), F32),
    mesh=_mesh,
    scratch_types=[
        pltpu.VMEM((NCHUNK, CH), jnp.int32),
        pltpu.VMEM((CH, 16), F32),
        pltpu.VMEM_SHARED((NPAD, 16), F32),
    ],
)
def _deg_kernel(dstp_hbm, ones_hbm, zeros_hbm, out_hbm, dst_v, ones_v, deg_sh):
    cid = lax.axis_index("c")
    sid = lax.axis_index("s")
    wid = cid * 16 + sid
    pltpu.sync_copy(dstp_hbm.at[wid], dst_v)
    pltpu.sync_copy(ones_hbm, ones_v)
    pltpu.sync_copy(zeros_hbm, deg_sh.at[pl.ds(sid * RPT, RPT)])
    plsc.subcore_barrier()

    def body(j, carry):
        pltpu.sync_copy(ones_v, deg_sh.at[dst_v.at[j]], add=True)
        return carry

    lax.fori_loop(0, NCHUNK, body, 0)
    plsc.subcore_barrier()
    pltpu.sync_copy(deg_sh.at[pl.ds(sid * RPT, RPT)],
                    out_hbm.at[cid, pl.ds(sid * RPT, RPT)])


@functools.partial(
    pl.kernel,
    out_type=jax.ShapeDtypeStruct((2, NPAD, D), F32),
    mesh=_mesh,
    scratch_types=[
        pltpu.VMEM((NCHUNK, CH), jnp.int32),
        pltpu.VMEM((NCHUNK, CH), jnp.int32),
        pltpu.VMEM((2, CH, D), F32),
        pltpu.VMEM_SHARED((NPAD, D), F32),
        pltpu.SemaphoreType.DMA,
        pltpu.SemaphoreType.DMA,
    ],
)
def _edge_kernel(y_hbm, srcp_hbm, dstp_hbm, zeros_hbm, out_hbm,
                 src_v, dst_v, rows_v, acc_sh, sem0, sem1):
    cid = lax.axis_index("c")
    sid = lax.axis_index("s")
    wid = cid * 16 + sid
    pltpu.sync_copy(srcp_hbm.at[wid], src_v)
    pltpu.sync_copy(dstp_hbm.at[wid], dst_v)
    pltpu.sync_copy(zeros_hbm, acc_sh.at[pl.ds(sid * RPT, RPT)])
    plsc.subcore_barrier()

    def gather(j, buf, sem):
        pltpu.async_copy(y_hbm.at[src_v.at[j]], rows_v.at[buf], sem)

    def gwait(j, buf, sem):
        pltpu.make_async_copy(y_hbm.at[src_v.at[j]], rows_v.at[buf], sem).wait()

    def scatter(j, buf):
        pltpu.sync_copy(rows_v.at[buf], acc_sh.at[dst_v.at[j]], add=True)

    gather(0, 0, sem0)
    gather(1, 1, sem1)

    def body(t, carry):
        j0 = 2 * t
        gwait(j0, 0, sem0)
        scatter(j0, 0)

        @pl.when(j0 + 2 < NCHUNK)
        def _():
            gather(j0 + 2, 0, sem0)

        gwait(j0 + 1, 1, sem1)
        scatter(j0 + 1, 1)

        @pl.when(j0 + 3 < NCHUNK)
        def _():
            gather(j0 + 3, 1, sem1)

        return carry

    lax.fori_loop(0, NCHUNK // 2, body, 0)
    plsc.subcore_barrier()
    pltpu.sync_copy(acc_sh.at[pl.ds(sid * RPT, RPT)],
                    out_hbm.at[cid, pl.ds(sid * RPT, RPT)])


def _dis(d0_ref, d1_ref):
    deg = d0_ref[:, 0:1] + d1_ref[:, 0:1] + 1.0
    return lax.rsqrt(deg)


def _mm(a, w_ref):
    return jnp.dot(a, w_ref[...], preferred_element_type=F32,
                   precision=lax.Precision.HIGHEST)


def _t1_body(x_ref, w_ref, d0_ref, d1_ref, o_ref):
    o_ref[...] = _mm(x_ref[...], w_ref) * _dis(d0_ref, d1_ref)


def _t2_body(p0_ref, p1_ref, y_ref, d0_ref, d1_ref, b_ref, w_ref, o_ref):
    dis = _dis(d0_ref, d1_ref)
    h = (p0_ref[...] + p1_ref[...] + y_ref[...]) * dis + b_ref[...]
    h = jnp.maximum(h, 0.0)
    o_ref[...] = _mm(h, w_ref) * dis


def _t3_body(p0_ref, p1_ref, y_ref, d0_ref, d1_ref, b_ref, o_ref):
    dis = _dis(d0_ref, d1_ref)
    o_ref[...] = (p0_ref[...] + p1_ref[...] + y_ref[...]) * dis + b_ref[...]


_spec_rows = pl.BlockSpec((R, D), lambda i: (i, 0))
_spec_w = pl.BlockSpec((D, D), lambda i: (0, 0))
_spec_b = pl.BlockSpec((1, D), lambda i: (0, 0))
_spec_d0 = pl.BlockSpec((None, R, 16), lambda i: (0, i, 0))
_spec_d1 = pl.BlockSpec((None, R, 16), lambda i: (1, i, 0))
_spec_p0 = pl.BlockSpec((None, R, D), lambda i: (0, i, 0))
_spec_p1 = pl.BlockSpec((None, R, D), lambda i: (1, i, 0))
_out_rows = jax.ShapeDtypeStruct((N, D), F32)


def _t1(x, W, degs):
    return pl.pallas_call(
        _t1_body, grid=(N // R,),
        in_specs=[_spec_rows, _spec_w, _spec_d0, _spec_d1],
        out_specs=_spec_rows, out_shape=_out_rows,
    )(x, W, degs, degs)


def _t2(parts, y, degs, b, W):
    return pl.pallas_call(
        _t2_body, grid=(N // R,),
        in_specs=[_spec_p0, _spec_p1, _spec_rows, _spec_d0, _spec_d1,
                  _spec_b, _spec_w],
        out_specs=_spec_rows, out_shape=_out_rows,
    )(parts, parts, y, degs, degs, b.reshape(1, D), W)


def _t3(parts, y, degs, b):
    return pl.pallas_call(
        _t3_body, grid=(N // R,),
        in_specs=[_spec_p0, _spec_p1, _spec_rows, _spec_d0, _spec_d1, _spec_b],
        out_specs=_spec_rows, out_shape=_out_rows,
    )(parts, parts, y, degs, degs, b.reshape(1, D))


def kernel(x, edge_index, W1, b1, W2, b2, W3, b3):
    src = edge_index[0].astype(jnp.int32)
    dst = edge_index[1].astype(jnp.int32)
    srcp = jnp.pad(src.reshape(NW, EPT),
                   ((0, 0), (0, EPTP - EPT))).reshape(NW, NCHUNK, CH)
    dstp = jnp.pad(dst.reshape(NW, EPT), ((0, 0), (0, EPTP - EPT)),
                   constant_values=N).reshape(NW, NCHUNK, CH)
    ones16 = jnp.ones((CH, 16), F32)
    zeros16 = jnp.zeros((RPT, 16), F32)
    zerosD = jnp.zeros((RPT, D), F32)

    degs = _deg_kernel(dstp, ones16, zeros16)
    y1 = _t1(x, W1, degs)
    e1 = _edge_kernel(y1, srcp, dstp, zerosD)
    y2 = _t2(e1, y1, degs, b1, W2)
    e2 = _edge_kernel(y2, srcp, dstp, zerosD)
    y3 = _t2(e2, y2, degs, b2, W3)
    e3 = _edge_kernel(y3, srcp, dstp, zerosD)
    return _t3(e3, y3, degs, b3)


# speedup vs baseline: 9.0186x; 9.0186x over previous
import functools

import jax
import jax.numpy as jnp
from jax import lax
from jax.experimental import pallas as pl
from jax.experimental.pallas import tpu as pltpu
from jax.experimental.pallas import tpu_sc as plsc

N = 10000
E = 320000
D = 128
NPAD = 10112
NW = 32
EPT = E // NW
CH = 128
NCHUNK = 80
KB = 8
NB = NCHUNK // KB
EPTP = NCHUNK * CH
RPT = NPAD // 16
R = 1000
F32 = jnp.float32

_mesh = plsc.VectorSubcoreMesh(core_axis_name="c", subcore_axis_name="s")


@functools.partial(
    pl.kernel,
    out_type=jax.ShapeDtypeStruct((2, NPAD, D), F32),
    mesh=_mesh,
    scratch_types=[
        pltpu.VMEM((NCHUNK, CH), jnp.int32),
        pltpu.VMEM((CH, D), F32),
        pltpu.VMEM_SHARED((NPAD, D), F32),
    ],
)
def _deg_kernel(dstp_hbm, ones_hbm, zeros_hbm, out_hbm, dst_v, ones_v, deg_sh):
    cid = lax.axis_index("c")
    sid = lax.axis_index("s")
    wid = cid * 16 + sid
    pltpu.sync_copy(dstp_hbm.at[wid], dst_v)
    pltpu.sync_copy(ones_hbm, ones_v)
    pltpu.sync_copy(zeros_hbm, deg_sh.at[pl.ds(sid * RPT, RPT)])
    plsc.subcore_barrier()

    def body(j, carry):
        pltpu.sync_copy(ones_v, deg_sh.at[dst_v.at[j]], add=True)
        return carry

    lax.fori_loop(0, NCHUNK, body, 0)
    plsc.subcore_barrier()
    pltpu.sync_copy(deg_sh.at[pl.ds(sid * RPT, RPT)],
                    out_hbm.at[cid, pl.ds(sid * RPT, RPT)])


@functools.partial(
    pl.kernel,
    out_type=jax.ShapeDtypeStruct((2, NPAD, D), F32),
    mesh=_mesh,
    scratch_types=[
        pltpu.VMEM((2, KB, CH), jnp.int32),
        pltpu.VMEM((NCHUNK, CH), jnp.int32),
        pltpu.VMEM((2, CH, D), F32),
        pltpu.VMEM_SHARED((NPAD, D), F32),
        pltpu.SemaphoreType.DMA,
        pltpu.SemaphoreType.DMA,
        pltpu.SemaphoreType.DMA,
    ],
)
def _edge_kernel(y_hbm, srcp_hbm, dstp_hbm, zeros_hbm, out_hbm,
                 sb_v, dst_v, rows_v, acc_sh, sem0, sem1, sem_i):
    cid = lax.axis_index("c")
    sid = lax.axis_index("s")
    wid = cid * 16 + sid
    pltpu.sync_copy(dstp_hbm.at[wid], dst_v)
    pltpu.sync_copy(zeros_hbm, acc_sh.at[pl.ds(sid * RPT, RPT)])

    def idx_copy(b, buf):
        return pltpu.make_async_copy(
            srcp_hbm.at[wid, pl.ds(b * KB, KB)], sb_v.at[buf], sem_i)

    def gather_copy(b, k, rbuf, sem):
        return pltpu.make_async_copy(
            y_hbm.at[sb_v.at[lax.rem(b, 2), k]], rows_v.at[rbuf], sem)

    def scatter(j, rbuf):
        pltpu.sync_copy(rows_v.at[rbuf], acc_sh.at[dst_v.at[j]], add=True)

    pltpu.sync_copy(srcp_hbm.at[wid, pl.ds(0, KB)], sb_v.at[0])
    idx_copy(1, 1).start()
    plsc.subcore_barrier()
    gather_copy(0, 0, 0, sem0).start()
    gather_copy(0, 1, 1, sem1).start()

    sems = (sem0, sem1)

    def body(b, carry):
        for k in range(KB):
            j = b * KB + k
            rbuf = k % 2
            gather_copy(b, k, rbuf, sems[rbuf]).wait()
            scatter(j, rbuf)
            if k == KB - 2:
                @pl.when(b + 1 < NB)
                def _():
                    idx_copy(b + 1, (b + 1) % 2).wait()
                    gather_copy(b + 1, 0, rbuf, sems[rbuf]).start()
            elif k == KB - 1:
                @pl.when(b + 1 < NB)
                def _():
                    gather_copy(b + 1, 1, rbuf, sems[rbuf]).start()
                @pl.when(b + 2 < NB)
                def _():
                    idx_copy(b + 2, b % 2).start()
            else:
                gather_copy(b, k + 2, rbuf, sems[rbuf]).start()
        return carry

    lax.fori_loop(0, NB, body, 0)
    plsc.subcore_barrier()
    pltpu.sync_copy(acc_sh.at[pl.ds(sid * RPT, RPT)],
                    out_hbm.at[cid, pl.ds(sid * RPT, RPT)])


def _dis(d0_ref, d1_ref):
    deg = d0_ref[:, 0:1] + d1_ref[:, 0:1] + 1.0
    return lax.rsqrt(deg)


def _mm(a, w_ref):
    return jnp.dot(a, w_ref[...], preferred_element_type=F32,
                   precision=lax.Precision.HIGHEST)


def _t1_body(x_ref, w_ref, d0_ref, d1_ref, o_ref):
    o_ref[...] = _mm(x_ref[...], w_ref) * _dis(d0_ref, d1_ref)


def _t2_body(p0_ref, p1_ref, y_ref, d0_ref, d1_ref, b_ref, w_ref, o_ref):
    dis = _dis(d0_ref, d1_ref)
    h = (p0_ref[...] + p1_ref[...] + y_ref[...]) * dis + b_ref[...]
    h = jnp.maximum(h, 0.0)
    o_ref[...] = _mm(h, w_ref) * dis


def _t3_body(p0_ref, p1_ref, y_ref, d0_ref, d1_ref, b_ref, o_ref):
    dis = _dis(d0_ref, d1_ref)
    o_ref[...] = (p0_ref[...] + p1_ref[...] + y_ref[...]) * dis + b_ref[...]


_spec_rows = pl.BlockSpec((R, D), lambda i: (i, 0))
_spec_w = pl.BlockSpec((D, D), lambda i: (0, 0))
_spec_b = pl.BlockSpec((1, D), lambda i: (0, 0))
_spec_d0 = pl.BlockSpec((None, R, D), lambda i: (0, i, 0))
_spec_d1 = pl.BlockSpec((None, R, D), lambda i: (1, i, 0))
_spec_p0 = pl.BlockSpec((None, R, D), lambda i: (0, i, 0))
_spec_p1 = pl.BlockSpec((None, R, D), lambda i: (1, i, 0))
_out_rows = jax.ShapeDtypeStruct((N, D), F32)


def _t1(x, W, degs):
    return pl.pallas_call(
        _t1_body, grid=(N // R,),
        in_specs=[_spec_rows, _spec_w, _spec_d0, _spec_d1],
        out_specs=_spec_rows, out_shape=_out_rows,
    )(x, W, degs, degs)


def _t2(parts, y, degs, b, W):
    return pl.pallas_call(
        _t2_body, grid=(N // R,),
        in_specs=[_spec_p0, _spec_p1, _spec_rows, _spec_d0, _spec_d1,
                  _spec_b, _spec_w],
        out_specs=_spec_rows, out_shape=_out_rows,
    )(parts, parts, y, degs, degs, b.reshape(1, D), W)


def _t3(parts, y, degs, b):
    return pl.pallas_call(
        _t3_body, grid=(N // R,),
        in_specs=[_spec_p0, _spec_p1, _spec_rows, _spec_d0, _spec_d1, _spec_b],
        out_specs=_spec_rows, out_shape=_out_rows,
    )(parts, parts, y, degs, degs, b.reshape(1, D))


def kernel(x, edge_index, W1, b1, W2, b2, W3, b3):
    src = edge_index[0].astype(jnp.int32)
    dst = edge_index[1].astype(jnp.int32)
    srcp = jnp.pad(src.reshape(NW, EPT),
                   ((0, 0), (0, EPTP - EPT))).reshape(NW, NCHUNK, CH)
    dstp = jnp.pad(dst.reshape(NW, EPT), ((0, 0), (0, EPTP - EPT)),
                   constant_values=N).reshape(NW, NCHUNK, CH)
    onesD = jnp.ones((CH, D), F32)
    zerosD = jnp.zeros((RPT, D), F32)

    degs = _deg_kernel(dstp, onesD, zerosD)
    y1 = _t1(x, W1, degs)
    e1 = _edge_kernel(y1, srcp, dstp, zerosD)
    y2 = _t2(e1, y1, degs, b1, W2)
    e2 = _edge_kernel(y2, srcp, dstp, zerosD)
    y3 = _t2(e2, y2, degs, b2, W3)
    e3 = _edge_kernel(y3, srcp, dstp, zerosD)
    return _t3(e3, y3, degs, b3)


# speedup vs baseline: 16.5391x; 1.8339x over previous
import functools

import jax
import jax.numpy as jnp
from jax import lax
from jax.experimental import pallas as pl
from jax.experimental.pallas import tpu as pltpu
from jax.experimental.pallas import tpu_sc as plsc

N = 10000
E = 320000
D = 128
NPAD = 10112
NW = 32
EPT = E // NW
CH = 120
NCHUNK = 84
KB = 6
NB = NCHUNK // KB
EPTP = NCHUNK * CH
RPT = NPAD // 16
R = 1000
F32 = jnp.float32

_mesh = plsc.VectorSubcoreMesh(core_axis_name="c", subcore_axis_name="s")


@functools.partial(
    pl.kernel,
    out_type=jax.ShapeDtypeStruct((2, NPAD, D), F32),
    mesh=_mesh,
    scratch_types=[
        pltpu.VMEM((NB, KB, CH), jnp.int32),
        pltpu.VMEM((CH, D), F32),
        pltpu.VMEM_SHARED((NPAD, D), F32),
    ],
)
def _deg_kernel(dstp_hbm, ones_hbm, zeros_hbm, out_hbm, dst_v, ones_v, deg_sh):
    cid = lax.axis_index("c")
    sid = lax.axis_index("s")
    wid = cid * 16 + sid
    pltpu.sync_copy(dstp_hbm.at[wid], dst_v)
    pltpu.sync_copy(ones_hbm, ones_v)
    pltpu.sync_copy(zeros_hbm, deg_sh.at[pl.ds(sid * RPT, RPT)])
    plsc.subcore_barrier()

    def body(b, carry):
        for k in range(KB):
            pltpu.sync_copy(ones_v, deg_sh.at[dst_v.at[b, k]], add=True)
        return carry

    lax.fori_loop(0, NB, body, 0)
    plsc.subcore_barrier()
    pltpu.sync_copy(deg_sh.at[pl.ds(sid * RPT, RPT)],
                    out_hbm.at[cid, pl.ds(sid * RPT, RPT)])


@functools.partial(
    pl.kernel,
    out_type=jax.ShapeDtypeStruct((2, NPAD, D), F32),
    mesh=_mesh,
    scratch_types=[
        pltpu.VMEM((2, KB, CH), jnp.int32),
        pltpu.VMEM((2, KB, CH), jnp.int32),
        pltpu.VMEM((3, CH, D), F32),
        pltpu.VMEM_SHARED((NPAD, D), F32),
        pltpu.SemaphoreType.DMA,
        pltpu.SemaphoreType.DMA,
        pltpu.SemaphoreType.DMA,
        pltpu.SemaphoreType.DMA,
    ],
)
def _edge_kernel(y_hbm, srcp_hbm, dstp_hbm, zeros_hbm, out_hbm,
                 sb_src, sb_dst, rows_v, acc_sh, g0, g1, g2, sem_i):
    cid = lax.axis_index("c")
    sid = lax.axis_index("s")
    wid = cid * 16 + sid
    gsems = (g0, g1, g2)

    def idx_copies(b, buf):
        return (pltpu.make_async_copy(srcp_hbm.at[wid, b], sb_src.at[buf], sem_i),
                pltpu.make_async_copy(dstp_hbm.at[wid, b], sb_dst.at[buf], sem_i))

    def idx_start(b, buf):
        for c in idx_copies(b, buf):
            c.start()

    def idx_wait(b, buf):
        for c in idx_copies(b, buf):
            c.wait()

    def gather_copy(bb, k, p):
        return pltpu.make_async_copy(
            y_hbm.at[sb_src.at[bb, k]], rows_v.at[p], gsems[p])

    def scatter(bb, k, p):
        pltpu.sync_copy(rows_v.at[p], acc_sh.at[sb_dst.at[bb, k]], add=True)

    pltpu.sync_copy(zeros_hbm, acc_sh.at[pl.ds(sid * RPT, RPT)])
    idx_start(0, 0)
    idx_wait(0, 0)
    idx_start(1, 1)
    plsc.subcore_barrier()
    gather_copy(0, 0, 0).start()
    gather_copy(0, 1, 1).start()
    gather_copy(0, 2, 2).start()

    def body(b, carry):
        bb = lax.rem(b, 2)
        bn = lax.rem(b + 1, 2)
        for k in range(KB):
            p = k % 3
            gather_copy(bb, k, p).wait()
            scatter(bb, k, p)
            if k == 2:
                @pl.when(b + 1 < NB)
                def _():
                    idx_wait(b + 1, bn)
            if k < 3:
                gather_copy(bb, k + 3, p).start()
            else:
                @pl.when(b + 1 < NB)
                def _():
                    gather_copy(bn, k - 3, p).start()
            if k == KB - 1:
                @pl.when(b + 2 < NB)
                def _():
                    idx_start(b + 2, bb)
        return carry

    lax.fori_loop(0, NB, body, 0)
    plsc.subcore_barrier()
    pltpu.sync_copy(acc_sh.at[pl.ds(sid * RPT, RPT)],
                    out_hbm.at[cid, pl.ds(sid * RPT, RPT)])


def _dis(d0_ref, d1_ref):
    deg = d0_ref[:, 0:1] + d1_ref[:, 0:1] + 1.0
    return lax.rsqrt(deg)


def _mm(a, w_ref):
    return jnp.dot(a, w_ref[...], preferred_element_type=F32,
                   precision=lax.Precision.HIGHEST)


def _t1_body(x_ref, w_ref, d0_ref, d1_ref, o_ref):
    o_ref[...] = _mm(x_ref[...], w_ref) * _dis(d0_ref, d1_ref)


def _t2_body(p0_ref, p1_ref, y_ref, d0_ref, d1_ref, b_ref, w_ref, o_ref):
    dis = _dis(d0_ref, d1_ref)
    h = (p0_ref[...] + p1_ref[...] + y_ref[...]) * dis + b_ref[...]
    h = jnp.maximum(h, 0.0)
    o_ref[...] = _mm(h, w_ref) * dis


def _t3_body(p0_ref, p1_ref, y_ref, d0_ref, d1_ref, b_ref, o_ref):
    dis = _dis(d0_ref, d1_ref)
    o_ref[...] = (p0_ref[...] + p1_ref[...] + y_ref[...]) * dis + b_ref[...]


_spec_rows = pl.BlockSpec((R, D), lambda i: (i, 0))
_spec_w = pl.BlockSpec((D, D), lambda i: (0, 0))
_spec_b = pl.BlockSpec((1, D), lambda i: (0, 0))
_spec_d0 = pl.BlockSpec((None, R, D), lambda i: (0, i, 0))
_spec_d1 = pl.BlockSpec((None, R, D), lambda i: (1, i, 0))
_spec_p0 = pl.BlockSpec((None, R, D), lambda i: (0, i, 0))
_spec_p1 = pl.BlockSpec((None, R, D), lambda i: (1, i, 0))
_out_rows = jax.ShapeDtypeStruct((N, D), F32)


def _t1(x, W, degs):
    return pl.pallas_call(
        _t1_body, grid=(N // R,),
        in_specs=[_spec_rows, _spec_w, _spec_d0, _spec_d1],
        out_specs=_spec_rows, out_shape=_out_rows,
    )(x, W, degs, degs)


def _t2(parts, y, degs, b, W):
    return pl.pallas_call(
        _t2_body, grid=(N // R,),
        in_specs=[_spec_p0, _spec_p1, _spec_rows, _spec_d0, _spec_d1,
                  _spec_b, _spec_w],
        out_specs=_spec_rows, out_shape=_out_rows,
    )(parts, parts, y, degs, degs, b.reshape(1, D), W)


def _t3(parts, y, degs, b):
    return pl.pallas_call(
        _t3_body, grid=(N // R,),
        in_specs=[_spec_p0, _spec_p1, _spec_rows, _spec_d0, _spec_d1, _spec_b],
        out_specs=_spec_rows, out_shape=_out_rows,
    )(parts, parts, y, degs, degs, b.reshape(1, D))


def kernel(x, edge_index, W1, b1, W2, b2, W3, b3):
    src = edge_index[0].astype(jnp.int32)
    dst = edge_index[1].astype(jnp.int32)
    srcp = jnp.pad(src.reshape(NW, EPT),
                   ((0, 0), (0, EPTP - EPT))).reshape(NW, NB, KB, CH)
    dstp = jnp.pad(dst.reshape(NW, EPT), ((0, 0), (0, EPTP - EPT)),
                   constant_values=N).reshape(NW, NB, KB, CH)
    onesD = jnp.ones((CH, D), F32)
    zerosD = jnp.zeros((RPT, D), F32)

    degs = _deg_kernel(dstp, onesD, zerosD)
    y1 = _t1(x, W1, degs)
    e1 = _edge_kernel(y1, srcp, dstp, zerosD)
    y2 = _t2(e1, y1, degs, b1, W2)
    e2 = _edge_kernel(y2, srcp, dstp, zerosD)
    y3 = _t2(e2, y2, degs, b2, W3)
    e3 = _edge_kernel(y3, srcp, dstp, zerosD)
    return _t3(e3, y3, degs, b3)
